# baseline (speedup 1.0000x reference)
import dataclasses
import functools

import jax
import jax.numpy as jnp
from jax import lax
from jax.experimental import pallas as pl
from jax.experimental.pallas import tpu as pltpu
from jax.experimental.pallas import tpu_sc as plsc

E = 320000
D = 128
NV = D // 16
N = 10000
NC = 2
NS = 16
EPT = E // (NC * NS)
CHUNK = 128
NBUF = 2
NFULL = EPT // CHUNK
TAIL = EPT - NFULL * CHUNK
STG = 128
ZROWS = 624
ZCHUNK = 104
ZTAIL = N - NS * ZROWS


def _sc_compiler_params():
    cp = pltpu.CompilerParams()
    if "needs_layout_passes" in pltpu.CompilerParams.__dataclass_fields__:
        cp = dataclasses.replace(cp, needs_layout_passes=False)
    return cp


def _sc_segment_sum(M, dest):
    mesh = plsc.VectorSubcoreMesh(core_axis_name="c", subcore_axis_name="s")

    @functools.partial(
        pl.kernel,
        out_type=jax.ShapeDtypeStruct((NC, N, D), jnp.float32),
        mesh=mesh,
        compiler_params=_sc_compiler_params(),
        scratch_types=[
            pltpu.VMEM((CHUNK, D), jnp.float32),
            pltpu.VMEM((CHUNK, D), jnp.float32),
            pltpu.VMEM((CHUNK + 16,), jnp.int32),
            pltpu.VMEM((CHUNK + 16,), jnp.int32),
            pltpu.VMEM((TAIL + 16,), jnp.int32),
            pltpu.VMEM((STG, D), jnp.float32),
            pltpu.VMEM((STG,), jnp.int32),
            pltpu.VMEM((D,), jnp.float32),
            pltpu.VMEM_SHARED((N, D), jnp.float32),
            pltpu.SemaphoreType.DMA,
            pltpu.SemaphoreType.DMA,
            pltpu.SemaphoreType.DMA,
            pltpu.SemaphoreType.DMA,
        ],
    )
    def k(m_hbm, d_hbm, out_hbm, mb0, mb1, ib0, ib1, itail, stage, istage,
          lastcum, acc, ms0, ms1, is0, is1):
        c = lax.axis_index("c")
        s = lax.axis_index("s")
        mbs = (mb0, mb1)
        ibs = (ib0, ib1)
        msems = (ms0, ms1)
        isems = (is0, is1)

        zero = jnp.zeros((16,), jnp.float32)

        @pl.loop(0, ZCHUNK)
        def _(r):
            @pl.loop(0, D, step=16)
            def _(col):
                mb0[r, pl.ds(col, 16)] = zero

        @pl.loop(0, ZROWS // ZCHUNK)
        def _(j):
            pltpu.sync_copy(
                mb0.at[pl.ds(0, ZCHUNK)],
                acc.at[pl.ds(s * ZROWS + j * ZCHUNK, ZCHUNK)],
            )

        @pl.when(s == NS - 1)
        def _():
            pltpu.sync_copy(
                mb0.at[pl.ds(0, ZTAIL)],
                acc.at[pl.ds(NS * ZROWS, ZTAIL)],
            )

        plsc.subcore_barrier()

        ebase = c * (NS * EPT) + s * EPT

        def start_in(j, b):
            off = ebase + j * CHUNK
            pltpu.async_copy(d_hbm.at[pl.ds(off, CHUNK)],
                             ibs[b].at[pl.ds(0, CHUNK)], isems[b])
            pltpu.async_copy(m_hbm.at[pl.ds(off, CHUNK)], mbs[b], msems[b])

        def wait_in(b):
            pltpu.make_async_copy(
                d_hbm.at[pl.ds(0, CHUNK)],
                ibs[b].at[pl.ds(0, CHUNK)], isems[b]).wait()
            pltpu.make_async_copy(
                m_hbm.at[pl.ds(0, CHUNK)], mbs[b], msems[b]).wait()

        for b in range(NBUF):
            start_in(b, b)

        lane = lax.iota(jnp.int32, 16)
        mask0 = lane == 0

        @pl.loop(0, D, step=16)
        def _(col):
            lastcum[pl.ds(col, 16)] = zero

        def flush(p, dcur, avecs):
            plsc.store_scatter(
                istage,
                [jnp.full((16,), p, jnp.int32)],
                jnp.full((16,), jnp.maximum(dcur, 0), jnp.int32),
                mask=mask0,
            )
            for v in range(NV):
                lc = lastcum[pl.ds(v * 16, 16)]
                stage[p, pl.ds(v * 16, 16)] = avecs[v] - lc
                lastcum[pl.ds(v * 16, 16)] = avecs[v]

        def process(ibref, mbref, nblk, carry):
            def block(blk, carry):
                e0 = blk * 16
                dv = ibref[pl.ds(e0, 16)]
                for i in range(16):
                    dcur, p, *avecs = carry
                    e = e0 + i
                    d = dv[i]
                    changed = d != dcur

                    @pl.when(changed)
                    def _(p=p, dcur=dcur, avecs=avecs):
                        flush(p, dcur, avecs)

                        @pl.when(p + 1 == STG)
                        def _():
                            pltpu.sync_copy(stage, acc.at[istage], add=True)

                    pn = jnp.where(changed,
                                   jnp.where(p + 1 == STG, 0, p + 1), p)
                    newa = [avecs[v] + mbref[e, pl.ds(v * 16, 16)]
                            for v in range(NV)]
                    carry = (d, pn, *newa)
                return carry

            return lax.fori_loop(0, nblk, block, carry)

        carry = (jnp.int32(-1), jnp.int32(0)) + tuple(
            jnp.zeros((16,), jnp.float32) for _ in range(NV))

        def outer(it, carry):
            j0 = it * NBUF
            for b in range(NBUF):
                wait_in(b)
                carry = process(ibs[b], mbs[b], CHUNK // 16, carry)

                @pl.when(j0 + b + NBUF < NFULL)
                def _(b=b):
                    start_in(j0 + b + NBUF, b)
            return carry

        carry = lax.fori_loop(0, NFULL // NBUF, outer, carry)

        toff = ebase + NFULL * CHUNK
        pltpu.sync_copy(d_hbm.at[pl.ds(toff, TAIL)], itail.at[pl.ds(0, TAIL)])
        pltpu.sync_copy(m_hbm.at[pl.ds(toff, TAIL)], mb0.at[pl.ds(0, TAIL)])
        carry = process(itail, mb0, TAIL // 16, carry)

        dcur, p, *avecs = carry
        flush(p, dcur, avecs)
        pfin = p + 1

        @pl.loop(0, STG, step=16)
        def _(g):
            cur = istage[pl.ds(g, 16)]
            istage[pl.ds(g, 16)] = jnp.where(lane + g >= pfin, 0, cur)

        @pl.loop(0, STG)
        def _(r):
            @pl.when(r >= pfin)
            def _():
                for v in range(NV):
                    stage[r, pl.ds(v * 16, 16)] = zero

        pltpu.sync_copy(stage, acc.at[istage], add=True)

        plsc.subcore_barrier()

        @pl.loop(0, ZROWS // ZCHUNK)
        def _(j):
            row = s * ZROWS + j * ZCHUNK
            pltpu.sync_copy(
                acc.at[pl.ds(row, ZCHUNK)],
                out_hbm.at[c].at[pl.ds(row, ZCHUNK)],
            )

        @pl.when(s == NS - 1)
        def _():
            pltpu.sync_copy(
                acc.at[pl.ds(NS * ZROWS, ZTAIL)],
                out_hbm.at[c].at[pl.ds(NS * ZROWS, ZTAIL)],
            )

    return k(M, dest)


def _tc_add_kernel(a_ref, b_ref, o_ref):
    o_ref[...] = a_ref[0] + b_ref[0]


def _tc_add(partials):
    blk = 1000
    return pl.pallas_call(
        _tc_add_kernel,
        out_shape=jax.ShapeDtypeStruct((N, D), jnp.float32),
        grid=(N // blk,),
        in_specs=[
            pl.BlockSpec((1, blk, D), lambda i: (0, i, 0)),
            pl.BlockSpec((1, blk, D), lambda i: (1, i, 0)),
        ],
        out_specs=pl.BlockSpec((blk, D), lambda i: (i, 0)),
    )(partials, partials)


def kernel(M, dest, dim_size):
    partials = _sc_segment_sum(M, dest.astype(jnp.int32))
    out = _tc_add(partials)
    w = jnp.ones((E, 1), dtype=M.dtype)
    return (out, w)


# speedup vs baseline: 4.0464x; 4.0464x over previous
import functools

import jax
import jax.numpy as jnp
from jax import lax
from jax.experimental import pallas as pl
from jax.experimental.pallas import tpu as pltpu
from jax.experimental.pallas import tpu_sc as plsc

E = 320000
D = 128
N = 10000
NC = 2
NS = 16
EPT = E // (NC * NS)
CHUNK = 128
NBUF = 3
NFULL = EPT // CHUNK
TAIL = EPT - NFULL * CHUNK
ZROWS = 624
ZCHUNK = 104
ZTAIL = N - NS * ZROWS


def _sc_segment_sum(M, dest):
    mesh = plsc.VectorSubcoreMesh(core_axis_name="c", subcore_axis_name="s")

    @functools.partial(
        pl.kernel,
        out_type=jax.ShapeDtypeStruct((NC, N, D), jnp.float32),
        mesh=mesh,
        scratch_types=[
            pltpu.VMEM((CHUNK, D), jnp.float32),
            pltpu.VMEM((CHUNK, D), jnp.float32),
            pltpu.VMEM((CHUNK, D), jnp.float32),
            pltpu.VMEM((CHUNK,), jnp.int32),
            pltpu.VMEM((CHUNK,), jnp.int32),
            pltpu.VMEM((CHUNK,), jnp.int32),
            pltpu.VMEM((TAIL,), jnp.int32),
            pltpu.VMEM_SHARED((N, D), jnp.float32),
            pltpu.SemaphoreType.DMA,
            pltpu.SemaphoreType.DMA,
            pltpu.SemaphoreType.DMA,
            pltpu.SemaphoreType.DMA,
            pltpu.SemaphoreType.DMA,
            pltpu.SemaphoreType.DMA,
        ],
    )
    def k(m_hbm, d_hbm, out_hbm, mb0, mb1, mb2, ib0, ib1, ib2, itail,
          acc, ms0, ms1, ms2, is0, is1, is2):
        c = lax.axis_index("c")
        s = lax.axis_index("s")
        mbs = (mb0, mb1, mb2)
        ibs = (ib0, ib1, ib2)
        msems = (ms0, ms1, ms2)
        isems = (is0, is1, is2)

        zero = jnp.zeros((16,), jnp.float32)

        @pl.loop(0, ZCHUNK)
        def _(r):
            @pl.loop(0, D, step=16)
            def _(col):
                mb0[r, pl.ds(col, 16)] = zero

        @pl.loop(0, ZROWS // ZCHUNK)
        def _(j):
            pltpu.sync_copy(
                mb0.at[pl.ds(0, ZCHUNK)],
                acc.at[pl.ds(s * ZROWS + j * ZCHUNK, ZCHUNK)],
            )

        @pl.when(s == NS - 1)
        def _():
            pltpu.sync_copy(
                mb0.at[pl.ds(0, ZTAIL)],
                acc.at[pl.ds(NS * ZROWS, ZTAIL)],
            )

        plsc.subcore_barrier()

        ebase = c * (NS * EPT) + s * EPT

        def start_in(j, b):
            off = ebase + j * CHUNK
            pltpu.async_copy(d_hbm.at[pl.ds(off, CHUNK)], ibs[b], isems[b])
            pltpu.async_copy(m_hbm.at[pl.ds(off, CHUNK)], mbs[b], msems[b])

        def wait_in(b):
            pltpu.make_async_copy(
                d_hbm.at[pl.ds(0, CHUNK)], ibs[b], isems[b]).wait()
            pltpu.make_async_copy(
                m_hbm.at[pl.ds(0, CHUNK)], mbs[b], msems[b]).wait()

        for b in range(NBUF):
            start_in(b, b)

        @pl.loop(0, NFULL // NBUF)
        def _(it):
            j0 = it * NBUF
            hs = []
            for b in range(NBUF):
                wait_in(b)
                hs.append(pltpu.async_copy(
                    mbs[b], acc.at[ibs[b]], msems[b], add=True))
            for b in range(NBUF):
                hs[b].wait()

                @pl.when(j0 + b + NBUF < NFULL)
                def _(b=b):
                    start_in(j0 + b + NBUF, b)

        toff = ebase + NFULL * CHUNK
        pltpu.sync_copy(d_hbm.at[pl.ds(toff, TAIL)], itail)
        pltpu.sync_copy(m_hbm.at[pl.ds(toff, TAIL)], mb0.at[pl.ds(0, TAIL)])
        pltpu.sync_copy(mb0.at[pl.ds(0, TAIL)], acc.at[itail], add=True)

        plsc.subcore_barrier()

        @pl.loop(0, ZROWS // ZCHUNK)
        def _(j):
            row = s * ZROWS + j * ZCHUNK
            pltpu.sync_copy(
                acc.at[pl.ds(row, ZCHUNK)],
                out_hbm.at[c].at[pl.ds(row, ZCHUNK)],
            )

        @pl.when(s == NS - 1)
        def _():
            pltpu.sync_copy(
                acc.at[pl.ds(NS * ZROWS, ZTAIL)],
                out_hbm.at[c].at[pl.ds(NS * ZROWS, ZTAIL)],
            )

    return k(M, dest)


def _tc_add_kernel(a_ref, b_ref, o_ref):
    o_ref[...] = a_ref[0] + b_ref[0]


def _tc_add(partials):
    blk = 1000
    return pl.pallas_call(
        _tc_add_kernel,
        out_shape=jax.ShapeDtypeStruct((N, D), jnp.float32),
        grid=(N // blk,),
        in_specs=[
            pl.BlockSpec((1, blk, D), lambda i: (0, i, 0)),
            pl.BlockSpec((1, blk, D), lambda i: (1, i, 0)),
        ],
        out_specs=pl.BlockSpec((blk, D), lambda i: (i, 0)),
    )(partials, partials)


def kernel(M, dest, dim_size):
    partials = _sc_segment_sum(M, dest.astype(jnp.int32))
    out = _tc_add(partials)
    w = jnp.ones((E, 1), dtype=M.dtype)
    return (out, w)


# speedup vs baseline: 4.1003x; 1.0133x over previous
import functools

import jax
import jax.numpy as jnp
from jax import lax
from jax.experimental import pallas as pl
from jax.experimental.pallas import tpu as pltpu
from jax.experimental.pallas import tpu_sc as plsc

E = 320000
D = 128
N = 10000
NC = 2
NS = 16
EPT = E // (NC * NS)
CHUNK = 128
NBUF = 3
NFULL = EPT // CHUNK
TAIL = EPT - NFULL * CHUNK
ZROWS = 624
ZCHUNK = 104
ZTAIL = N - NS * ZROWS


def _sc_segment_sum(M, dest):
    mesh = plsc.VectorSubcoreMesh(core_axis_name="c", subcore_axis_name="s")

    @functools.partial(
        pl.kernel,
        out_type=jax.ShapeDtypeStruct((NC, N, D), jnp.float32),
        mesh=mesh,
        scratch_types=[
            pltpu.VMEM((CHUNK, D), jnp.float32),
            pltpu.VMEM((CHUNK, D), jnp.float32),
            pltpu.VMEM((CHUNK, D), jnp.float32),
            pltpu.VMEM((CHUNK,), jnp.int32),
            pltpu.VMEM((CHUNK,), jnp.int32),
            pltpu.VMEM((CHUNK,), jnp.int32),
            pltpu.VMEM((TAIL,), jnp.int32),
            pltpu.VMEM_SHARED((N, D), jnp.float32),
            pltpu.SemaphoreType.DMA,
            pltpu.SemaphoreType.DMA,
            pltpu.SemaphoreType.DMA,
            pltpu.SemaphoreType.DMA,
            pltpu.SemaphoreType.DMA,
            pltpu.SemaphoreType.DMA,
        ],
    )
    def k(m_hbm, d_hbm, out_hbm, mb0, mb1, mb2, ib0, ib1, ib2, itail,
          acc, ms0, ms1, ms2, is0, is1, is2):
        c = lax.axis_index("c")
        s = lax.axis_index("s")
        mbs = (mb0, mb1, mb2)
        ibs = (ib0, ib1, ib2)
        msems = (ms0, ms1, ms2)
        isems = (is0, is1, is2)

        ebase = c * (NS * EPT) + s * EPT

        def start_in(j, b):
            off = ebase + j * CHUNK
            pltpu.async_copy(d_hbm.at[pl.ds(off, CHUNK)], ibs[b], isems[b])
            pltpu.async_copy(m_hbm.at[pl.ds(off, CHUNK)], mbs[b], msems[b])

        def wait_in(b):
            pltpu.make_async_copy(
                d_hbm.at[pl.ds(0, CHUNK)], ibs[b], isems[b]).wait()
            pltpu.make_async_copy(
                m_hbm.at[pl.ds(0, CHUNK)], mbs[b], msems[b]).wait()

        start_in(0, 1)
        start_in(1, 2)

        zero = jnp.zeros((16,), jnp.float32)

        @pl.loop(0, ZCHUNK)
        def _(r):
            @pl.loop(0, D, step=16)
            def _(col):
                mb0[r, pl.ds(col, 16)] = zero

        @pl.loop(0, ZROWS // ZCHUNK)
        def _(j):
            pltpu.async_copy(
                mb0.at[pl.ds(0, ZCHUNK)],
                acc.at[pl.ds(s * ZROWS + j * ZCHUNK, ZCHUNK)],
                ms0,
            )

        @pl.when(s == NS - 1)
        def _():
            pltpu.async_copy(
                mb0.at[pl.ds(0, ZTAIL)],
                acc.at[pl.ds(NS * ZROWS, ZTAIL)],
                ms0,
            )

        @pl.loop(0, ZROWS // ZCHUNK)
        def _(j):
            pltpu.make_async_copy(
                mb0.at[pl.ds(0, ZCHUNK)],
                acc.at[pl.ds(0, ZCHUNK)],
                ms0,
            ).wait()

        @pl.when(s == NS - 1)
        def _():
            pltpu.make_async_copy(
                mb0.at[pl.ds(0, ZTAIL)],
                acc.at[pl.ds(0, ZTAIL)],
                ms0,
            ).wait()

        plsc.subcore_barrier()

        start_in(2, 0)

        RB = (1, 2, 0)

        @pl.loop(0, NFULL // NBUF)
        def _(it):
            j0 = it * NBUF
            hs = []
            for k in range(NBUF):
                b = RB[k]
                wait_in(b)
                hs.append(pltpu.async_copy(
                    mbs[b], acc.at[ibs[b]], msems[b], add=True))
            for k in range(NBUF):
                hs[k].wait()

                @pl.when(j0 + k + NBUF < NFULL)
                def _(k=k):
                    start_in(j0 + k + NBUF, RB[k])

        toff = ebase + NFULL * CHUNK
        pltpu.sync_copy(d_hbm.at[pl.ds(toff, TAIL)], itail)
        pltpu.sync_copy(m_hbm.at[pl.ds(toff, TAIL)], mb0.at[pl.ds(0, TAIL)])
        pltpu.sync_copy(mb0.at[pl.ds(0, TAIL)], acc.at[itail], add=True)

        plsc.subcore_barrier()

        @pl.loop(0, ZROWS // ZCHUNK)
        def _(j):
            row = s * ZROWS + j * ZCHUNK
            pltpu.async_copy(
                acc.at[pl.ds(row, ZCHUNK)],
                out_hbm.at[c].at[pl.ds(row, ZCHUNK)],
                ms0,
            )

        @pl.when(s == NS - 1)
        def _():
            pltpu.async_copy(
                acc.at[pl.ds(NS * ZROWS, ZTAIL)],
                out_hbm.at[c].at[pl.ds(NS * ZROWS, ZTAIL)],
                ms0,
            )

        @pl.loop(0, ZROWS // ZCHUNK)
        def _(j):
            pltpu.make_async_copy(
                acc.at[pl.ds(0, ZCHUNK)],
                out_hbm.at[c].at[pl.ds(0, ZCHUNK)],
                ms0,
            ).wait()

        @pl.when(s == NS - 1)
        def _():
            pltpu.make_async_copy(
                acc.at[pl.ds(0, ZTAIL)],
                out_hbm.at[c].at[pl.ds(0, ZTAIL)],
                ms0,
            ).wait()

    return k(M, dest)


def _tc_add_kernel(a_ref, b_ref, o_ref):
    o_ref[...] = a_ref[0] + b_ref[0]


def _tc_add(partials):
    blk = 1000
    return pl.pallas_call(
        _tc_add_kernel,
        out_shape=jax.ShapeDtypeStruct((N, D), jnp.float32),
        grid=(N // blk,),
        in_specs=[
            pl.BlockSpec((1, blk, D), lambda i: (0, i, 0)),
            pl.BlockSpec((1, blk, D), lambda i: (1, i, 0)),
        ],
        out_specs=pl.BlockSpec((blk, D), lambda i: (i, 0)),
    )(partials, partials)


def kernel(M, dest, dim_size):
    partials = _sc_segment_sum(M, dest.astype(jnp.int32))
    out = _tc_add(partials)
    w = jnp.ones((E, 1), dtype=M.dtype)
    return (out, w)
